# Initial kernel scaffold; baseline (speedup 1.0000x reference)
import jax
import jax.numpy as jnp
from jax.experimental import pallas as pl


def kernel(x, edge_attr, ring_attr, parent_index, edge_index, ring_node_ring, ring_node_node, node_batch, edge_batch, ring_batch, Wn, bn0, We, be0, Wr, br0, Ws, Wc, bbu, gbu, bbn_bu, A1, A2, ab1, A3, ab3, ge, bge, R1, R2, rb1, R3, rb3, gr, bgr, P1, P2, P3, pb1, gp, bgp, Wo, bo):
    raise NotImplementedError("write your pallas kernel here")


# speedup vs baseline: 2.4939x; 2.4939x over previous
import functools

import jax
import jax.numpy as jnp
from jax import lax
from jax.experimental import pallas as pl
from jax.experimental.pallas import tpu as pltpu
from jax.experimental.pallas import tpu_sc as plsc

EPS = 1e-5
NC, NS = 2, 16
NW = NC * NS
CH = 128
B = 64
H = 64


def _relu(x):
    return jnp.maximum(x, 0.0)


def _sc_mesh():
    return plsc.VectorSubcoreMesh(
        core_axis_name="c", subcore_axis_name="s",
        num_cores=NC, num_subcores=NS)


def _seg_acc(acc_ref, ids, xval, t):
    chk = 250
    for j in range(t // chk):
        idj = ids[:, j * chk:(j + 1) * chk]
        rows = lax.broadcasted_iota(jnp.int32, (B, chk), 0)
        ohj = (rows == idj).astype(jnp.float32)
        acc_ref[...] += jnp.dot(ohj, xval[j * chk:(j + 1) * chk, :],
                                preferred_element_type=jnp.float32,
                                precision=lax.Precision.HIGHEST)


def _init_body(x_ref, w_ref, b_ref, o_ref):
    h0 = _relu(jnp.dot(x_ref[...], w_ref[...],
                       preferred_element_type=jnp.float32) + b_ref[...])
    o_ref[...] = jnp.concatenate(
        [h0, h0], axis=1)


def _init_h(x, w, b, tile):
    n, d = x.shape
    return pl.pallas_call(
        _init_body,
        grid=(n // tile,),
        in_specs=[
            pl.BlockSpec((tile, d), lambda i: (i, 0)),
            pl.BlockSpec((d, H), lambda i: (0, 0)),
            pl.BlockSpec((1, H), lambda i: (0, 0)),
        ],
        out_specs=pl.BlockSpec((tile, 2 * H), lambda i: (i, 0)),
        out_shape=jax.ShapeDtypeStruct((n, 2 * H), jnp.float32),
    )(x, w, b)


def _bu_body(h_ref, ch_ref, ws_ref, wc_ref, b_ref, y_ref, st_ref, acc1, acc2):
    i = pl.program_id(0)
    y = (jnp.dot(h_ref[...][:, :H], ws_ref[...],
                 preferred_element_type=jnp.float32)
         + jnp.dot(ch_ref[0][:, :H], wc_ref[...],
                   preferred_element_type=jnp.float32)
         + b_ref[...])
    y_ref[...] = y

    @pl.when(i == 0)
    def _():
        acc1[...] = jnp.zeros_like(acc1)
        acc2[...] = jnp.zeros_like(acc2)

    acc1[...] += jnp.sum(y, axis=0, keepdims=True)
    acc2[...] += jnp.sum(y * y, axis=0, keepdims=True)

    @pl.when(i == pl.num_programs(0) - 1)
    def _():
        st_ref[...] = jnp.concatenate(
            [acc1[...], acc2[...], jnp.zeros((6, H), jnp.float32)], axis=0)


def _bottom_up(hh, ch4, ws, wc, b, tile, rng_rows):
    n = hh.shape[0]
    tpr = rng_rows // tile
    return pl.pallas_call(
        _bu_body,
        grid=(n // tile,),
        in_specs=[
            pl.BlockSpec((tile, 2 * H), lambda i: (i, 0)),
            pl.BlockSpec((1, tile, 2 * H), lambda i: (i // tpr, i % tpr, 0)),
            pl.BlockSpec((H, H), lambda i: (0, 0)),
            pl.BlockSpec((H, H), lambda i: (0, 0)),
            pl.BlockSpec((1, H), lambda i: (0, 0)),
        ],
        out_specs=[
            pl.BlockSpec((tile, H), lambda i: (i, 0)),
            pl.BlockSpec((8, H), lambda i: (0, 0)),
        ],
        out_shape=[
            jax.ShapeDtypeStruct((n, H), jnp.float32),
            jax.ShapeDtypeStruct((8, H), jnp.float32),
        ],
        scratch_shapes=[
            pltpu.VMEM((1, H), jnp.float32),
            pltpu.VMEM((1, H), jnp.float32),
        ],
    )(hh, ch4, ws, wc, b)


def _bn_relu_body(y_ref, st_ref, g_ref, b_ref, o_ref, *, count):
    m = st_ref[0:1, :] / count
    v = st_ref[1:2, :] / count - m * m
    h = _relu((y_ref[...] - m) * lax.rsqrt(v + EPS) * g_ref[...] + b_ref[...])
    o_ref[...] = jnp.concatenate(
        [h, h], axis=1)


def _bn_relu(y, st, g, b, tile, count):
    n = y.shape[0]
    return pl.pallas_call(
        functools.partial(_bn_relu_body, count=count),
        grid=(n // tile,),
        in_specs=[
            pl.BlockSpec((tile, H), lambda i: (i, 0)),
            pl.BlockSpec((8, H), lambda i: (0, 0)),
            pl.BlockSpec((1, H), lambda i: (0, 0)),
            pl.BlockSpec((1, H), lambda i: (0, 0)),
        ],
        out_specs=pl.BlockSpec((tile, 2 * H), lambda i: (i, 0)),
        out_shape=jax.ShapeDtypeStruct((n, 2 * H), jnp.float32),
    )(y, st, g, b)


def _top_body(top_ref, ids_ref, agg_ref, acc):
    i = pl.program_id(0)
    t = top_ref[...][:, :H]

    @pl.when(i == 0)
    def _():
        acc[...] = jnp.zeros_like(acc)

    _seg_acc(acc, ids_ref[0], t, t.shape[0])

    @pl.when(i == pl.num_programs(0) - 1)
    def _():
        agg_ref[...] = acc[...]


def _top_post(top, ids3, tile):
    n = top.shape[0]
    return pl.pallas_call(
        _top_body,
        grid=(n // tile,),
        in_specs=[
            pl.BlockSpec((tile, 2 * H), lambda i: (i, 0)),
            pl.BlockSpec((1, 1, tile), lambda i: (i, 0, 0)),
        ],
        out_specs=pl.BlockSpec((B, H), lambda i: (0, 0)),
        out_shape=jax.ShapeDtypeStruct((B, H), jnp.float32),
        scratch_shapes=[pltpu.VMEM((B, H), jnp.float32)],
    )(top, ids3)


def _mlp(attr, gs, goff, w, b0, m1, m2, mb1, m3, mb3, tile):
    n, de = attr.shape
    ng = len(gs)
    gw = gs[0].shape[1]

    def body(*refs):
        at_ref = refs[0]
        g_refs = refs[1:1 + ng]
        (w_ref, b0_ref, m1_ref, m2_ref, mb1_ref, m3_ref, mb3_ref,
         me_ref, st_ref, acc1, acc2) = refs[1 + ng:]
        i = pl.program_id(0)
        gsum = g_refs[0][...][:, goff:goff + H]
        for gr_ in g_refs[1:]:
            gsum = gsum + gr_[...][:, goff:goff + H]
        a0 = _relu(jnp.dot(at_ref[...], w_ref[...],
                           preferred_element_type=jnp.float32) + b0_ref[...])
        z = (jnp.dot(a0, m1_ref[...], preferred_element_type=jnp.float32)
             + jnp.dot(gsum, m2_ref[...], preferred_element_type=jnp.float32)
             + mb1_ref[...])
        me = jnp.dot(_relu(z), m3_ref[...],
                     preferred_element_type=jnp.float32) + mb3_ref[...]
        me_ref[...] = me

        @pl.when(i == 0)
        def _():
            acc1[...] = jnp.zeros_like(acc1)
            acc2[...] = jnp.zeros_like(acc2)

        acc1[...] += jnp.sum(me, axis=0, keepdims=True)
        acc2[...] += jnp.sum(me * me, axis=0, keepdims=True)

        @pl.when(i == pl.num_programs(0) - 1)
        def _():
            st_ref[...] = jnp.concatenate(
                [acc1[...], acc2[...], jnp.zeros((6, H), jnp.float32)], axis=0)

    return pl.pallas_call(
        body,
        grid=(n // tile,),
        in_specs=[pl.BlockSpec((tile, de), lambda i: (i, 0))]
        + [pl.BlockSpec((tile, gw), lambda i: (i, 0)) for _ in range(ng)]
        + [
            pl.BlockSpec((de, H), lambda i: (0, 0)),
            pl.BlockSpec((1, H), lambda i: (0, 0)),
            pl.BlockSpec((H, H), lambda i: (0, 0)),
            pl.BlockSpec((H, H), lambda i: (0, 0)),
            pl.BlockSpec((1, H), lambda i: (0, 0)),
            pl.BlockSpec((H, H), lambda i: (0, 0)),
            pl.BlockSpec((1, H), lambda i: (0, 0)),
        ],
        out_specs=[
            pl.BlockSpec((tile, H), lambda i: (i, 0)),
            pl.BlockSpec((8, H), lambda i: (0, 0)),
        ],
        out_shape=[
            jax.ShapeDtypeStruct((n, H), jnp.float32),
            jax.ShapeDtypeStruct((8, H), jnp.float32),
        ],
        scratch_shapes=[
            pltpu.VMEM((1, H), jnp.float32),
            pltpu.VMEM((1, H), jnp.float32),
        ],
    )(attr, *gs, w, b0, m1, m2, mb1, m3, mb3)


def _bnagg_body(me_ref, st_ref, g_ref, b_ref, ids_ref, agg_ref, acc, *, count):
    i = pl.program_id(0)
    m = st_ref[0:1, :] / count
    v = st_ref[1:2, :] / count - m * m
    rep = _relu((me_ref[...] - m) * lax.rsqrt(v + EPS) * g_ref[...] + b_ref[...])

    @pl.when(i == 0)
    def _():
        acc[...] = jnp.zeros_like(acc)

    _seg_acc(acc, ids_ref[0], rep, rep.shape[0])

    @pl.when(i == pl.num_programs(0) - 1)
    def _():
        agg_ref[...] = acc[...]


def _bnagg(me, st, g, b, ids3, tile, count):
    n = me.shape[0]
    return pl.pallas_call(
        functools.partial(_bnagg_body, count=count),
        grid=(n // tile,),
        in_specs=[
            pl.BlockSpec((tile, H), lambda i: (i, 0)),
            pl.BlockSpec((8, H), lambda i: (0, 0)),
            pl.BlockSpec((1, H), lambda i: (0, 0)),
            pl.BlockSpec((1, H), lambda i: (0, 0)),
            pl.BlockSpec((1, 1, tile), lambda i: (i, 0, 0)),
        ],
        out_specs=pl.BlockSpec((B, H), lambda i: (0, 0)),
        out_shape=jax.ShapeDtypeStruct((B, H), jnp.float32),
        scratch_shapes=[pltpu.VMEM((B, H), jnp.float32)],
    )(me, st, g, b, ids3)


def _final_body(na_ref, ea_ref, ra_ref, p1_ref, p2_ref, p3_ref, pb_ref,
                g_ref, b_ref, wo_ref, bo_ref, o_ref):
    s = (jnp.dot(na_ref[...], p1_ref[...], preferred_element_type=jnp.float32)
         + jnp.dot(ea_ref[...], p2_ref[...], preferred_element_type=jnp.float32)
         + jnp.dot(ra_ref[...], p3_ref[...], preferred_element_type=jnp.float32)
         + pb_ref[...])
    m = jnp.mean(s, axis=0, keepdims=True)
    v = jnp.mean(s * s, axis=0, keepdims=True) - m * m
    hp = _relu((s - m) * lax.rsqrt(v + EPS) * g_ref[...] + b_ref[...])
    o_ref[...] = jnp.dot(hp, wo_ref[...],
                         preferred_element_type=jnp.float32) + bo_ref[...]


def _final(na, ea, ra, p1, p2, p3, pb, g, b, wo_pad, bo_pad):
    return pl.pallas_call(
        _final_body,
        out_shape=jax.ShapeDtypeStruct((B, 128), jnp.float32),
    )(na, ea, ra, p1, p2, p3, pb, g, b, wo_pad, bo_pad)


NRANGE = 4
RNG_ROWS = 12800


def _parent_scatter(hh, pid, zrows):
    n = hh.shape[0]
    gw = hh.shape[1]
    rpw = zrows.shape[0]
    spr = rpw * NS
    nfull = n // CH
    tail = n - nfull * CH
    kmain = nfull // NS
    rem = nfull - kmain * NS

    def body(h_hbm, pid_hbm, z_hbm, out_hbm, idx_v, idx2_v, idx_t, idx2_t,
             rows_v, rows_t, spmem):
        c = lax.axis_index("c")
        s = lax.axis_index("s")

        def remap(src, dst, nn, base):
            for j in range(nn // 16):
                v = src[pl.ds(j * 16, 16)] - base
                ok = (v >= 0) & (v < RNG_ROWS)
                dst[pl.ds(j * 16, 16)] = jnp.where(ok, v, RNG_ROWS)

        for j in range(NRANGE // NC):
            t = c * (NRANGE // NC) + j
            base = t * RNG_ROWS

            pltpu.sync_copy(z_hbm, spmem.at[pl.ds(s * rpw, rpw), :])
            plsc.subcore_barrier()

            def chunk(ci):
                off = ci * CH
                pltpu.sync_copy(pid_hbm.at[pl.ds(off, CH)], idx_v)
                pltpu.sync_copy(h_hbm.at[pl.ds(off, CH), :], rows_v)
                remap(idx_v, idx2_v, CH, base)
                pltpu.sync_copy(rows_v, spmem.at[idx2_v], add=True)

            def loop(k, carry):
                chunk(s + k * NS)
                return carry

            lax.fori_loop(0, kmain, loop, 0)

            @pl.when(s < rem)
            def _():
                chunk(kmain * NS + s)

            if tail:
                @pl.when(s == rem)
                def _():
                    off = nfull * CH
                    pltpu.sync_copy(pid_hbm.at[pl.ds(off, tail)], idx_t)
                    pltpu.sync_copy(h_hbm.at[pl.ds(off, tail), :], rows_t)
                    remap(idx_t, idx2_t, tail, base)
                    pltpu.sync_copy(rows_t, spmem.at[idx2_t], add=True)

            plsc.subcore_barrier()
            pltpu.sync_copy(spmem.at[pl.ds(s * rpw, rpw), :],
                            out_hbm.at[t, pl.ds(s * rpw, rpw), :])
            plsc.subcore_barrier()

    return pl.kernel(
        body,
        out_type=jax.ShapeDtypeStruct((NRANGE, spr, gw), jnp.float32),
        mesh=_sc_mesh(),
        scratch_types=[
            pltpu.VMEM((CH,), jnp.int32),
            pltpu.VMEM((CH,), jnp.int32),
            pltpu.VMEM((max(tail, 16),), jnp.int32),
            pltpu.VMEM((max(tail, 16),), jnp.int32),
            pltpu.VMEM((CH, gw), jnp.float32),
            pltpu.VMEM((max(tail, 16), gw), jnp.float32),
            pltpu.VMEM_SHARED((spr, gw), jnp.float32),
        ],
    )(hh, pid, zrows)


def _edge_gather(pk, ei0, ei1):
    e = ei0.shape[0]
    gw = pk.shape[1]
    nchunks = e // CH
    kmain = nchunks // NW
    rem = nchunks - kmain * NW

    def body(pk_hbm, i0_hbm, i1_hbm, g_hbm, i0_v, i1_v, rows_v, sem):
        c = lax.axis_index("c")
        s = lax.axis_index("s")
        w = s * NC + c

        def chunk(ci):
            off = ci * CH
            pltpu.sync_copy(i0_hbm.at[pl.ds(off, CH)], i0_v)
            pltpu.sync_copy(i1_hbm.at[pl.ds(off, CH)], i1_v)
            pltpu.async_copy(pk_hbm.at[i0_v], rows_v, sem).wait()
            pltpu.async_copy(pk_hbm.at[i1_v], rows_v, sem, add=True).wait()
            pltpu.sync_copy(rows_v, g_hbm.at[pl.ds(off, CH), :])

        def loop(k, carry):
            chunk(w + k * NW)
            return carry

        lax.fori_loop(0, kmain, loop, 0)

        @pl.when(w < rem)
        def _():
            chunk(kmain * NW + w)

    return pl.kernel(
        body,
        out_type=jax.ShapeDtypeStruct((e, gw), jnp.float32),
        mesh=_sc_mesh(),
        scratch_types=[
            pltpu.VMEM((CH,), jnp.int32),
            pltpu.VMEM((CH,), jnp.int32),
            pltpu.VMEM((CH, gw), jnp.float32),
            pltpu.SemaphoreType.DMA,
        ],
    )(pk, ei0, ei1)


def _ring_gather_scatter(pk, rnn, rnr, zrows):
    gw = pk.shape[1]
    spr = zrows.shape[0] * NS
    nr = rnn.shape[0]
    rpw = spr // NS
    nfull = nr // CH
    tail = nr - nfull * CH
    kmain = nfull // NW
    rem = nfull - kmain * NW

    def body(t_hbm, rnn_hbm, rnr_hbm, z_hbm, out_hbm,
             idxn, idxr, idxn_t, idxr_t, rows_v, rows_t, spmem, sem):
        c = lax.axis_index("c")
        s = lax.axis_index("s")
        w = s * NC + c

        pltpu.sync_copy(z_hbm, spmem.at[pl.ds(s * rpw, rpw), :])
        plsc.subcore_barrier()

        def chunk(ci):
            off = ci * CH
            pltpu.sync_copy(rnn_hbm.at[pl.ds(off, CH)], idxn)
            pltpu.sync_copy(rnr_hbm.at[pl.ds(off, CH)], idxr)
            pltpu.async_copy(t_hbm.at[idxn], rows_v, sem).wait()
            pltpu.sync_copy(rows_v, spmem.at[idxr], add=True)

        def loop(k, carry):
            chunk(w + k * NW)
            return carry

        lax.fori_loop(0, kmain, loop, 0)

        @pl.when(w < rem)
        def _():
            chunk(kmain * NW + w)

        if tail:
            @pl.when(w == rem)
            def _():
                off = nfull * CH
                pltpu.sync_copy(rnn_hbm.at[pl.ds(off, tail)], idxn_t)
                pltpu.sync_copy(rnr_hbm.at[pl.ds(off, tail)], idxr_t)
                pltpu.async_copy(t_hbm.at[idxn_t], rows_t, sem).wait()
                pltpu.sync_copy(rows_t, spmem.at[idxr_t], add=True)

        plsc.subcore_barrier()
        pltpu.sync_copy(spmem.at[pl.ds(s * rpw, rpw), :],
                        out_hbm.at[c, pl.ds(s * rpw, rpw), :])

    return pl.kernel(
        body,
        out_type=jax.ShapeDtypeStruct((NC, spr, gw), jnp.float32),
        mesh=_sc_mesh(),
        scratch_types=[
            pltpu.VMEM((CH,), jnp.int32),
            pltpu.VMEM((CH,), jnp.int32),
            pltpu.VMEM((max(tail, 16),), jnp.int32),
            pltpu.VMEM((max(tail, 16),), jnp.int32),
            pltpu.VMEM((CH, gw), jnp.float32),
            pltpu.VMEM((max(tail, 16), gw), jnp.float32),
            pltpu.VMEM_SHARED((spr, gw), jnp.float32),
            pltpu.SemaphoreType.DMA,
        ],
    )(pk, rnn, rnr, zrows)


def kernel(x, edge_attr, ring_attr, parent_index, edge_index, ring_node_ring,
           ring_node_node, node_batch, edge_batch, ring_batch, Wn, bn0, We,
           be0, Wr, br0, Ws, Wc, bbu, gbu, bbn_bu, A1, A2, ab1, A3, ab3, ge,
           bge, R1, R2, rb1, R3, rb3, gr, bgr, P1, P2, P3, pb1, gp, bgp, Wo,
           bo):
    n = x.shape[0]
    e = edge_attr.shape[0]
    r = ring_attr.shape[0]

    tn, te, tr = 2000, 4000, 2000

    f32 = jnp.float32
    row = lambda v: v.reshape(1, -1).astype(f32)
    nb_i = node_batch.reshape(n // tn, 1, tn)
    eb_i = edge_batch.reshape(e // te, 1, te)
    rb_i = ring_batch.reshape(r // tr, 1, tr)
    ei0 = edge_index[0]
    ei1 = edge_index[1]
    wo_pad = jnp.pad(Wo, ((0, 0), (0, 128 - Wo.shape[1])))
    bo_pad = jnp.pad(bo, (0, 128 - bo.shape[0])).reshape(1, 128)
    zn = jnp.zeros((808, 2 * H), f32)
    zr = jnp.zeros((640, 2 * H), f32)

    hh = _init_h(x, Wn, row(bn0), tn)
    for _ in range(3):
        ch4 = _parent_scatter(hh, parent_index, zn)
        y, st = _bottom_up(hh, ch4, Ws, Wc, row(bbu), 400, RNG_ROWS)
        hh = _bn_relu(y, st, row(gbu), row(bbn_bu), tn, float(n))

    nagg = _top_post(hh, nb_i, tn)

    g = _edge_gather(hh, ei0, ei1)
    me, est = _mlp(edge_attr, [g], 0, We, row(be0), A1, A2, row(ab1), A3,
                   row(ab3), te)
    eagg = _bnagg(me, est, row(ge), row(bge), eb_i, te, float(e))

    rsum = _ring_gather_scatter(hh, ring_node_node, ring_node_ring, zr)
    mr, rst = _mlp(ring_attr, [rsum[0, :r], rsum[1, :r]], 0, Wr, row(br0),
                   R1, R2, row(rb1), R3, row(rb3), tr)
    ragg = _bnagg(mr, rst, row(gr), row(bgr), rb_i, tr, float(r))

    out = _final(nagg, eagg, ragg, P1, P2, P3, row(pb1), row(gp), row(bgp),
                 wo_pad, bo_pad)
    return out[:, :1]


# speedup vs baseline: 2.6456x; 1.0608x over previous
import functools

import jax
import jax.numpy as jnp
from jax import lax
from jax.experimental import pallas as pl
from jax.experimental.pallas import tpu as pltpu
from jax.experimental.pallas import tpu_sc as plsc

EPS = 1e-5
NC, NS = 2, 16
NW = NC * NS
CH = 128
B = 64
H = 64


def _relu(x):
    return jnp.maximum(x, 0.0)


def _sc_mesh():
    return plsc.VectorSubcoreMesh(
        core_axis_name="c", subcore_axis_name="s",
        num_cores=NC, num_subcores=NS)


def _seg_acc(acc_ref, ids, xval, t):
    chk = 250
    for j in range(t // chk):
        idj = ids[:, j * chk:(j + 1) * chk]
        rows = lax.broadcasted_iota(jnp.int32, (B, chk), 0)
        ohj = (rows == idj).astype(jnp.float32)
        acc_ref[...] += jnp.dot(ohj, xval[j * chk:(j + 1) * chk, :],
                                preferred_element_type=jnp.float32,
                                precision=lax.Precision.HIGHEST)


def _init_body(x_ref, w_ref, b_ref, o_ref):
    h0 = _relu(jnp.dot(x_ref[...], w_ref[...],
                       preferred_element_type=jnp.float32) + b_ref[...])
    o_ref[...] = jnp.concatenate(
        [h0, h0], axis=1)


def _init_h(x, w, b, tile):
    n, d = x.shape
    return pl.pallas_call(
        _init_body,
        grid=(n // tile,),
        in_specs=[
            pl.BlockSpec((tile, d), lambda i: (i, 0)),
            pl.BlockSpec((d, H), lambda i: (0, 0)),
            pl.BlockSpec((1, H), lambda i: (0, 0)),
        ],
        out_specs=pl.BlockSpec((tile, 2 * H), lambda i: (i, 0)),
        out_shape=jax.ShapeDtypeStruct((n, 2 * H), jnp.float32),
    )(x, w, b)


def _bu_body(h_ref, ch_ref, ws_ref, wc_ref, b_ref, y_ref, st_ref, acc1, acc2):
    i = pl.program_id(0)
    y = (jnp.dot(h_ref[...][:, :H], ws_ref[...],
                 preferred_element_type=jnp.float32)
         + jnp.dot(ch_ref[0][:, :H], wc_ref[...],
                   preferred_element_type=jnp.float32)
         + b_ref[...])
    y_ref[...] = y

    @pl.when(i == 0)
    def _():
        acc1[...] = jnp.zeros_like(acc1)
        acc2[...] = jnp.zeros_like(acc2)

    acc1[...] += jnp.sum(y, axis=0, keepdims=True)
    acc2[...] += jnp.sum(y * y, axis=0, keepdims=True)

    @pl.when(i == pl.num_programs(0) - 1)
    def _():
        st_ref[...] = jnp.concatenate(
            [acc1[...], acc2[...], jnp.zeros((6, H), jnp.float32)], axis=0)


def _bottom_up(hh, ch4, ws, wc, b, tile, rng_rows):
    n = hh.shape[0]
    tpr = rng_rows // tile
    return pl.pallas_call(
        _bu_body,
        grid=(n // tile,),
        in_specs=[
            pl.BlockSpec((tile, 2 * H), lambda i: (i, 0)),
            pl.BlockSpec((1, tile, 2 * H), lambda i: (i // tpr, i % tpr, 0)),
            pl.BlockSpec((H, H), lambda i: (0, 0)),
            pl.BlockSpec((H, H), lambda i: (0, 0)),
            pl.BlockSpec((1, H), lambda i: (0, 0)),
        ],
        out_specs=[
            pl.BlockSpec((tile, H), lambda i: (i, 0)),
            pl.BlockSpec((8, H), lambda i: (0, 0)),
        ],
        out_shape=[
            jax.ShapeDtypeStruct((n, H), jnp.float32),
            jax.ShapeDtypeStruct((8, H), jnp.float32),
        ],
        scratch_shapes=[
            pltpu.VMEM((1, H), jnp.float32),
            pltpu.VMEM((1, H), jnp.float32),
        ],
    )(hh, ch4, ws, wc, b)


def _bn_relu_body(y_ref, st_ref, g_ref, b_ref, o_ref, *, count):
    m = st_ref[0:1, :] / count
    v = st_ref[1:2, :] / count - m * m
    h = _relu((y_ref[...] - m) * lax.rsqrt(v + EPS) * g_ref[...] + b_ref[...])
    o_ref[...] = jnp.concatenate(
        [h, h], axis=1)


def _bn_relu(y, st, g, b, tile, count):
    n = y.shape[0]
    return pl.pallas_call(
        functools.partial(_bn_relu_body, count=count),
        grid=(n // tile,),
        in_specs=[
            pl.BlockSpec((tile, H), lambda i: (i, 0)),
            pl.BlockSpec((8, H), lambda i: (0, 0)),
            pl.BlockSpec((1, H), lambda i: (0, 0)),
            pl.BlockSpec((1, H), lambda i: (0, 0)),
        ],
        out_specs=pl.BlockSpec((tile, 2 * H), lambda i: (i, 0)),
        out_shape=jax.ShapeDtypeStruct((n, 2 * H), jnp.float32),
    )(y, st, g, b)


def _top_body(top_ref, ids_ref, agg_ref, acc):
    i = pl.program_id(0)
    t = top_ref[...][:, :H]

    @pl.when(i == 0)
    def _():
        acc[...] = jnp.zeros_like(acc)

    _seg_acc(acc, ids_ref[0], t, t.shape[0])

    @pl.when(i == pl.num_programs(0) - 1)
    def _():
        agg_ref[...] = acc[...]


def _top_post(top, ids3, tile):
    n = top.shape[0]
    return pl.pallas_call(
        _top_body,
        grid=(n // tile,),
        in_specs=[
            pl.BlockSpec((tile, 2 * H), lambda i: (i, 0)),
            pl.BlockSpec((1, 1, tile), lambda i: (i, 0, 0)),
        ],
        out_specs=pl.BlockSpec((B, H), lambda i: (0, 0)),
        out_shape=jax.ShapeDtypeStruct((B, H), jnp.float32),
        scratch_shapes=[pltpu.VMEM((B, H), jnp.float32)],
    )(top, ids3)


def _mlp(attr, gs, goff, w, b0, m1, m2, mb1, m3, mb3, tile):
    n, de = attr.shape
    ng = len(gs)
    gw = gs[0].shape[1]

    def body(*refs):
        at_ref = refs[0]
        g_refs = refs[1:1 + ng]
        (w_ref, b0_ref, m1_ref, m2_ref, mb1_ref, m3_ref, mb3_ref,
         me_ref, st_ref, acc1, acc2) = refs[1 + ng:]
        i = pl.program_id(0)
        gsum = g_refs[0][...][:, goff:goff + H]
        for gr_ in g_refs[1:]:
            gsum = gsum + gr_[...][:, goff:goff + H]
        a0 = _relu(jnp.dot(at_ref[...], w_ref[...],
                           preferred_element_type=jnp.float32) + b0_ref[...])
        z = (jnp.dot(a0, m1_ref[...], preferred_element_type=jnp.float32)
             + jnp.dot(gsum, m2_ref[...], preferred_element_type=jnp.float32)
             + mb1_ref[...])
        me = jnp.dot(_relu(z), m3_ref[...],
                     preferred_element_type=jnp.float32) + mb3_ref[...]
        me_ref[...] = me

        @pl.when(i == 0)
        def _():
            acc1[...] = jnp.zeros_like(acc1)
            acc2[...] = jnp.zeros_like(acc2)

        acc1[...] += jnp.sum(me, axis=0, keepdims=True)
        acc2[...] += jnp.sum(me * me, axis=0, keepdims=True)

        @pl.when(i == pl.num_programs(0) - 1)
        def _():
            st_ref[...] = jnp.concatenate(
                [acc1[...], acc2[...], jnp.zeros((6, H), jnp.float32)], axis=0)

    return pl.pallas_call(
        body,
        grid=(n // tile,),
        in_specs=[pl.BlockSpec((tile, de), lambda i: (i, 0))]
        + [pl.BlockSpec((tile, gw), lambda i: (i, 0)) for _ in range(ng)]
        + [
            pl.BlockSpec((de, H), lambda i: (0, 0)),
            pl.BlockSpec((1, H), lambda i: (0, 0)),
            pl.BlockSpec((H, H), lambda i: (0, 0)),
            pl.BlockSpec((H, H), lambda i: (0, 0)),
            pl.BlockSpec((1, H), lambda i: (0, 0)),
            pl.BlockSpec((H, H), lambda i: (0, 0)),
            pl.BlockSpec((1, H), lambda i: (0, 0)),
        ],
        out_specs=[
            pl.BlockSpec((tile, H), lambda i: (i, 0)),
            pl.BlockSpec((8, H), lambda i: (0, 0)),
        ],
        out_shape=[
            jax.ShapeDtypeStruct((n, H), jnp.float32),
            jax.ShapeDtypeStruct((8, H), jnp.float32),
        ],
        scratch_shapes=[
            pltpu.VMEM((1, H), jnp.float32),
            pltpu.VMEM((1, H), jnp.float32),
        ],
    )(attr, *gs, w, b0, m1, m2, mb1, m3, mb3)


def _bnagg_body(me_ref, st_ref, g_ref, b_ref, ids_ref, agg_ref, acc, *, count):
    i = pl.program_id(0)
    m = st_ref[0:1, :] / count
    v = st_ref[1:2, :] / count - m * m
    rep = _relu((me_ref[...] - m) * lax.rsqrt(v + EPS) * g_ref[...] + b_ref[...])

    @pl.when(i == 0)
    def _():
        acc[...] = jnp.zeros_like(acc)

    _seg_acc(acc, ids_ref[0], rep, rep.shape[0])

    @pl.when(i == pl.num_programs(0) - 1)
    def _():
        agg_ref[...] = acc[...]


def _bnagg(me, st, g, b, ids3, tile, count):
    n = me.shape[0]
    return pl.pallas_call(
        functools.partial(_bnagg_body, count=count),
        grid=(n // tile,),
        in_specs=[
            pl.BlockSpec((tile, H), lambda i: (i, 0)),
            pl.BlockSpec((8, H), lambda i: (0, 0)),
            pl.BlockSpec((1, H), lambda i: (0, 0)),
            pl.BlockSpec((1, H), lambda i: (0, 0)),
            pl.BlockSpec((1, 1, tile), lambda i: (i, 0, 0)),
        ],
        out_specs=pl.BlockSpec((B, H), lambda i: (0, 0)),
        out_shape=jax.ShapeDtypeStruct((B, H), jnp.float32),
        scratch_shapes=[pltpu.VMEM((B, H), jnp.float32)],
    )(me, st, g, b, ids3)


def _final_body(na_ref, ea_ref, ra_ref, p1_ref, p2_ref, p3_ref, pb_ref,
                g_ref, b_ref, wo_ref, bo_ref, o_ref):
    s = (jnp.dot(na_ref[...], p1_ref[...], preferred_element_type=jnp.float32)
         + jnp.dot(ea_ref[...], p2_ref[...], preferred_element_type=jnp.float32)
         + jnp.dot(ra_ref[...], p3_ref[...], preferred_element_type=jnp.float32)
         + pb_ref[...])
    m = jnp.mean(s, axis=0, keepdims=True)
    v = jnp.mean(s * s, axis=0, keepdims=True) - m * m
    hp = _relu((s - m) * lax.rsqrt(v + EPS) * g_ref[...] + b_ref[...])
    o_ref[...] = jnp.dot(hp, wo_ref[...],
                         preferred_element_type=jnp.float32) + bo_ref[...]


def _final(na, ea, ra, p1, p2, p3, pb, g, b, wo_pad, bo_pad):
    return pl.pallas_call(
        _final_body,
        out_shape=jax.ShapeDtypeStruct((B, 128), jnp.float32),
    )(na, ea, ra, p1, p2, p3, pb, g, b, wo_pad, bo_pad)


NRANGE = 4
RNG_ROWS = 12800


def _parent_scatter(hh, pid, zrows):
    n = hh.shape[0]
    gw = hh.shape[1]
    rpw = zrows.shape[0]
    spr = rpw * NS
    nfull = n // CH
    tail = n - nfull * CH
    kmain = nfull // NS
    rem = nfull - kmain * NS

    def body(h_hbm, pid_hbm, z_hbm, out_hbm, idx_v, idx2_v, idx_t, idx2_t,
             rows_v, rows_t, spmem):
        c = lax.axis_index("c")
        s = lax.axis_index("s")

        def remap(src, dst, nn, base):
            for j in range(nn // 16):
                v = src[pl.ds(j * 16, 16)] - base
                ok = (v >= 0) & (v < RNG_ROWS)
                dst[pl.ds(j * 16, 16)] = jnp.where(ok, v, RNG_ROWS)

        for j in range(NRANGE // NC):
            t = c * (NRANGE // NC) + j
            base = t * RNG_ROWS

            pltpu.sync_copy(z_hbm, spmem.at[pl.ds(s * rpw, rpw), :])
            plsc.subcore_barrier()

            def chunk(ci):
                off = ci * CH
                pltpu.sync_copy(pid_hbm.at[pl.ds(off, CH)], idx_v)
                pltpu.sync_copy(h_hbm.at[pl.ds(off, CH), :], rows_v)
                remap(idx_v, idx2_v, CH, base)
                pltpu.sync_copy(rows_v, spmem.at[idx2_v], add=True)

            def loop(k, carry):
                chunk(s + k * NS)
                return carry

            lax.fori_loop(0, kmain, loop, 0)

            @pl.when(s < rem)
            def _():
                chunk(kmain * NS + s)

            if tail:
                @pl.when(s == rem)
                def _():
                    off = nfull * CH
                    pltpu.sync_copy(pid_hbm.at[pl.ds(off, tail)], idx_t)
                    pltpu.sync_copy(h_hbm.at[pl.ds(off, tail), :], rows_t)
                    remap(idx_t, idx2_t, tail, base)
                    pltpu.sync_copy(rows_t, spmem.at[idx2_t], add=True)

            plsc.subcore_barrier()
            pltpu.sync_copy(spmem.at[pl.ds(s * rpw, rpw), :],
                            out_hbm.at[t, pl.ds(s * rpw, rpw), :])
            plsc.subcore_barrier()

    return pl.kernel(
        body,
        out_type=jax.ShapeDtypeStruct((NRANGE, spr, gw), jnp.float32),
        mesh=_sc_mesh(),
        scratch_types=[
            pltpu.VMEM((CH,), jnp.int32),
            pltpu.VMEM((CH,), jnp.int32),
            pltpu.VMEM((max(tail, 16),), jnp.int32),
            pltpu.VMEM((max(tail, 16),), jnp.int32),
            pltpu.VMEM((CH, gw), jnp.float32),
            pltpu.VMEM((max(tail, 16), gw), jnp.float32),
            pltpu.VMEM_SHARED((spr, gw), jnp.float32),
        ],
    )(hh, pid, zrows)


def _edge_gather(pk, ei0, ei1):
    e = ei0.shape[0]
    gw = pk.shape[1]
    nchunks = e // CH
    kmain = nchunks // NW
    rem = nchunks - kmain * NW

    def body(pk_hbm, i0_hbm, i1_hbm, g_hbm, i0_v, i1_v, rows_v, sem):
        c = lax.axis_index("c")
        s = lax.axis_index("s")
        w = s * NC + c
        nk = kmain + (w < rem).astype(jnp.int32)

        def start_first(k, b):
            off = (w + k * NW) * CH
            pltpu.sync_copy(i0_hbm.at[pl.ds(off, CH)], i0_v.at[b])
            pltpu.sync_copy(i1_hbm.at[pl.ds(off, CH)], i1_v.at[b])
            pltpu.make_async_copy(pk_hbm.at[i0_v.at[b]], rows_v.at[b],
                                  sem.at[b]).start()

        start_first(0, 0)

        def loop(k, carry):
            b = lax.rem(k, 2)
            nb = 1 - b

            @pl.when(k + 1 < nk)
            def _():
                start_first(k + 1, nb)

            pltpu.make_async_copy(pk_hbm.at[i0_v.at[b]], rows_v.at[b],
                                  sem.at[b]).wait()
            pltpu.async_copy(pk_hbm.at[i1_v.at[b]], rows_v.at[b], sem.at[b],
                             add=True).wait()
            off = (w + k * NW) * CH
            pltpu.sync_copy(rows_v.at[b], g_hbm.at[pl.ds(off, CH), :])
            return carry

        lax.fori_loop(0, nk, loop, 0)

    return pl.kernel(
        body,
        out_type=jax.ShapeDtypeStruct((e, gw), jnp.float32),
        mesh=_sc_mesh(),
        scratch_types=[
            pltpu.VMEM((2, CH), jnp.int32),
            pltpu.VMEM((2, CH), jnp.int32),
            pltpu.VMEM((2, CH, gw), jnp.float32),
            pltpu.SemaphoreType.DMA((2,)),
        ],
    )(pk, ei0, ei1)


def _ring_gather_scatter(pk, rnn, rnr, zrows):
    gw = pk.shape[1]
    spr = zrows.shape[0] * NS
    nr = rnn.shape[0]
    rpw = spr // NS
    nfull = nr // CH
    tail = nr - nfull * CH
    kmain = nfull // NW
    rem = nfull - kmain * NW

    def body(t_hbm, rnn_hbm, rnr_hbm, z_hbm, out_hbm,
             idxn, idxr, idxn_t, idxr_t, rows_v, rows_t, spmem, sem):
        c = lax.axis_index("c")
        s = lax.axis_index("s")
        w = s * NC + c

        pltpu.sync_copy(z_hbm, spmem.at[pl.ds(s * rpw, rpw), :])
        plsc.subcore_barrier()

        def chunk(ci):
            off = ci * CH
            pltpu.sync_copy(rnn_hbm.at[pl.ds(off, CH)], idxn)
            pltpu.sync_copy(rnr_hbm.at[pl.ds(off, CH)], idxr)
            pltpu.async_copy(t_hbm.at[idxn], rows_v, sem).wait()
            pltpu.sync_copy(rows_v, spmem.at[idxr], add=True)

        def loop(k, carry):
            chunk(w + k * NW)
            return carry

        lax.fori_loop(0, kmain, loop, 0)

        @pl.when(w < rem)
        def _():
            chunk(kmain * NW + w)

        if tail:
            @pl.when(w == rem)
            def _():
                off = nfull * CH
                pltpu.sync_copy(rnn_hbm.at[pl.ds(off, tail)], idxn_t)
                pltpu.sync_copy(rnr_hbm.at[pl.ds(off, tail)], idxr_t)
                pltpu.async_copy(t_hbm.at[idxn_t], rows_t, sem).wait()
                pltpu.sync_copy(rows_t, spmem.at[idxr_t], add=True)

        plsc.subcore_barrier()
        pltpu.sync_copy(spmem.at[pl.ds(s * rpw, rpw), :],
                        out_hbm.at[c, pl.ds(s * rpw, rpw), :])

    return pl.kernel(
        body,
        out_type=jax.ShapeDtypeStruct((NC, spr, gw), jnp.float32),
        mesh=_sc_mesh(),
        scratch_types=[
            pltpu.VMEM((CH,), jnp.int32),
            pltpu.VMEM((CH,), jnp.int32),
            pltpu.VMEM((max(tail, 16),), jnp.int32),
            pltpu.VMEM((max(tail, 16),), jnp.int32),
            pltpu.VMEM((CH, gw), jnp.float32),
            pltpu.VMEM((max(tail, 16), gw), jnp.float32),
            pltpu.VMEM_SHARED((spr, gw), jnp.float32),
            pltpu.SemaphoreType.DMA,
        ],
    )(pk, rnn, rnr, zrows)


def kernel(x, edge_attr, ring_attr, parent_index, edge_index, ring_node_ring,
           ring_node_node, node_batch, edge_batch, ring_batch, Wn, bn0, We,
           be0, Wr, br0, Ws, Wc, bbu, gbu, bbn_bu, A1, A2, ab1, A3, ab3, ge,
           bge, R1, R2, rb1, R3, rb3, gr, bgr, P1, P2, P3, pb1, gp, bgp, Wo,
           bo):
    n = x.shape[0]
    e = edge_attr.shape[0]
    r = ring_attr.shape[0]

    tn, te, tr = 2000, 4000, 2000

    f32 = jnp.float32
    row = lambda v: v.reshape(1, -1).astype(f32)
    nb_i = node_batch.reshape(n // tn, 1, tn)
    eb_i = edge_batch.reshape(e // te, 1, te)
    rb_i = ring_batch.reshape(r // tr, 1, tr)
    ei0 = edge_index[0]
    ei1 = edge_index[1]
    wo_pad = jnp.pad(Wo, ((0, 0), (0, 128 - Wo.shape[1])))
    bo_pad = jnp.pad(bo, (0, 128 - bo.shape[0])).reshape(1, 128)
    zn = jnp.zeros((808, 2 * H), f32)
    zr = jnp.zeros((640, 2 * H), f32)

    hh = _init_h(x, Wn, row(bn0), tn)
    for _ in range(3):
        ch4 = _parent_scatter(hh, parent_index, zn)
        y, st = _bottom_up(hh, ch4, Ws, Wc, row(bbu), 400, RNG_ROWS)
        hh = _bn_relu(y, st, row(gbu), row(bbn_bu), tn, float(n))

    nagg = _top_post(hh, nb_i, tn)

    g = _edge_gather(hh, ei0, ei1)
    me, est = _mlp(edge_attr, [g], 0, We, row(be0), A1, A2, row(ab1), A3,
                   row(ab3), te)
    eagg = _bnagg(me, est, row(ge), row(bge), eb_i, te, float(e))

    rsum = _ring_gather_scatter(hh, ring_node_node, ring_node_ring, zr)
    mr, rst = _mlp(ring_attr, [rsum[0, :r], rsum[1, :r]], 0, Wr, row(br0),
                   R1, R2, row(rb1), R3, row(rb3), tr)
    ragg = _bnagg(mr, rst, row(gr), row(bgr), rb_i, tr, float(r))

    out = _final(nagg, eagg, ragg, P1, P2, P3, row(pb1), row(gp), row(bgp),
                 wo_pad, bo_pad)
    return out[:, :1]
